# baseline (speedup 1.0000x reference)
import jax
import jax.numpy as jnp
from jax.experimental import pallas as pl
from jax.experimental.pallas import tpu as pltpu

_L = 2048
_X, _Y = 8, 4


def _body(w_ref, h_ref, o_ref):
    i = pl.program_id(0)

    @pl.when(i == 0)
    def _():
        o_ref[...] = jnp.zeros_like(o_ref)

    s = jnp.sum(w_ref[...] * h_ref[...], axis=2)
    o_ref[...] += s.T


@jax.jit
def _run(wT, hT):
    o48 = pl.pallas_call(
        _body,
        grid=(4,),
        in_specs=[
            pl.BlockSpec((_X, _Y, _L // 4), lambda i: (0, 0, i)),
            pl.BlockSpec((_X, _Y, _L // 4), lambda i: (0, 0, i)),
        ],
        out_specs=pl.BlockSpec((_Y, _X), lambda i: (0, 0)),
        out_shape=jax.ShapeDtypeStruct((_Y, _X), jnp.float32),
        compiler_params=pltpu.CompilerParams(
            dimension_semantics=("arbitrary",)
        ),
    )(wT, hT)
    return jnp.transpose(o48)


def kernel(x, adj, W_att, a_att, W_out):
    hT = jnp.transpose(x[0], (1, 2, 0))
    wT = jnp.transpose(W_out, (1, 2, 0))
    return _run(wT, hT)


# speedup vs baseline: 1.5920x; 1.5920x over previous
import jax
import jax.numpy as jnp
from jax.experimental import pallas as pl

_L = 2048
_X, _Y = 8, 4


def _body(w_ref, h_ref, o_ref):
    s = jnp.sum(w_ref[...] * h_ref[...], axis=2)
    o_ref[...] = s.T


@jax.jit
def _run(wT, hT):
    o_yx = pl.pallas_call(
        _body,
        out_shape=jax.ShapeDtypeStruct((_Y, _X), jnp.float32),
    )(wT, hT)
    return jnp.transpose(o_yx)


def kernel(x, adj, W_att, a_att, W_out):
    hT = jnp.transpose(x[0], (1, 2, 0))
    wT = jnp.transpose(W_out, (1, 2, 0))
    return _run(wT, hT)
